# initial kernel scaffold (unmeasured)
import jax
import jax.numpy as jnp
from jax import lax
from jax.experimental import pallas as pl
from jax.experimental.pallas import tpu as pltpu

N_DEV = 4
SQ = 1024
D = 1024
HQ = 8
DH = 128
SCALE = 0.08838834764831843


def _rot_cols(w):
    w2 = w.reshape(w.shape[0], -1, 2)
    return jnp.stack([-w2[..., 1], w2[..., 0]], axis=-1).reshape(w.shape)


def kernel(x, Wq, Wk, Wv, Wo):
    bf16 = jnp.bfloat16
    x = x.reshape(SQ, D).astype(bf16)
    Wq = Wq.astype(bf16)
    Wk = Wk.astype(bf16)
    Wv = Wv.astype(bf16)
    Wo = Wo.astype(bf16)
    WqP = _rot_cols(Wq)
    WkP = _rot_cols(Wk)

    inv = 10000.0 ** (-jnp.arange(0, DH, 2, dtype=jnp.float32) / DH)
    ang = jnp.arange(SQ, dtype=jnp.float32)[:, None] * inv[None, :]
    cos = jnp.tile(jnp.repeat(jnp.cos(ang), 2, axis=-1), (1, HQ)).astype(bf16)
    sin = jnp.tile(jnp.repeat(jnp.sin(ang), 2, axis=-1), (1, HQ)).astype(bf16)

    def body(x_ref, wq_ref, wk_ref, wv_ref, wqp_ref, wkp_ref, wo_ref,
             cos_ref, sin_ref, out_ref,
             xall, psend, acc, x_send_sems, x_recv_sems,
             p_send_sems, p_recv_sems):
        me = lax.axis_index("i")

        barrier = pltpu.get_barrier_semaphore()
        for r in range(1, N_DEV):
            pl.semaphore_signal(
                barrier, inc=1,
                device_id=((me + r) % N_DEV,),
                device_id_type=pl.DeviceIdType.MESH,
            )
        pl.semaphore_wait(barrier, N_DEV - 1)

        x_sends = []
        for r in range(1, N_DEV):
            rdma = pltpu.make_async_remote_copy(
                src_ref=x_ref,
                dst_ref=xall.at[3 - r],
                send_sem=x_send_sems.at[r - 1],
                recv_sem=x_recv_sems.at[3 - r],
                device_id=((me + r) % N_DEV,),
                device_id_type=pl.DeviceIdType.MESH,
            )
            rdma.start()
            x_sends.append(rdma)

        def mm(a, b):
            return lax.dot_general(
                a, b, (((1,), (0,)), ((), ())),
                preferred_element_type=jnp.float32,
            )

        def partial_for(xb):
            cosv = cos_ref[...].astype(jnp.float32)
            sinv = sin_ref[...].astype(jnp.float32)
            q = (mm(xb, wq_ref[...]) * cosv + mm(xb, wqp_ref[...]) * sinv).astype(bf16)
            k = (mm(xb, wk_ref[...]) * cosv + mm(xb, wkp_ref[...]) * sinv).astype(bf16)
            v = mm(xb, wv_ref[...]).astype(bf16)
            out = None
            for h in range(HQ):
                sl = slice(h * DH, (h + 1) * DH)
                s = lax.dot_general(
                    q[:, sl], k[:, sl], (((1,), (1,)), ((), ())),
                    preferred_element_type=jnp.float32,
                ) * SCALE
                m = jnp.max(s, axis=-1, keepdims=True)
                e = jnp.exp(s - m)
                den = jnp.sum(e, axis=-1, keepdims=True)
                ctx = (mm(e.astype(bf16), v[:, sl]) / den).astype(bf16)
                ph = mm(ctx, wo_ref[sl, :])
                out = ph if out is None else out + ph
            return out

        out_ref[0, :, :] = partial_for(x_ref[...])

        p_sends = []
        for s in range(N_DEV - 1):
            recv = pltpu.make_async_remote_copy(
                src_ref=x_ref,
                dst_ref=xall.at[s],
                send_sem=x_send_sems.at[0],
                recv_sem=x_recv_sems.at[s],
                device_id=(me,),
                device_id_type=pl.DeviceIdType.MESH,
            )
            recv.wait_recv()
            psend[s, :, :] = partial_for(xall[s]).astype(bf16)
            rdma = pltpu.make_async_remote_copy(
                src_ref=psend.at[s],
                dst_ref=acc.at[2 - s],
                send_sem=p_send_sems.at[s],
                recv_sem=p_recv_sems.at[2 - s],
                device_id=((me + s + 1) % N_DEV,),
                device_id_type=pl.DeviceIdType.MESH,
            )
            rdma.start()
            p_sends.append(rdma)

        for s in range(N_DEV - 1):
            recv = pltpu.make_async_remote_copy(
                src_ref=psend.at[0],
                dst_ref=acc.at[s],
                send_sem=p_send_sems.at[0],
                recv_sem=p_recv_sems.at[s],
                device_id=(me,),
                device_id_type=pl.DeviceIdType.MESH,
            )
            recv.wait_recv()

        out_ref[0, :, :] = out_ref[0, :, :] + (
            acc[0].astype(jnp.float32)
            + acc[1].astype(jnp.float32)
            + acc[2].astype(jnp.float32)
        )

        for rdma in x_sends + p_sends:
            rdma.wait_send()

    return pl.pallas_call(
        body,
        out_shape=jax.ShapeDtypeStruct((1, SQ, D), jnp.float32),
        in_specs=[pl.BlockSpec(memory_space=pltpu.VMEM)] * 9,
        out_specs=pl.BlockSpec(memory_space=pltpu.VMEM),
        scratch_shapes=[
            pltpu.VMEM((N_DEV - 1, SQ, D), jnp.bfloat16),
            pltpu.VMEM((N_DEV - 1, SQ, D), jnp.bfloat16),
            pltpu.VMEM((N_DEV - 1, SQ, D), jnp.bfloat16),
            pltpu.SemaphoreType.DMA((N_DEV - 1,)),
            pltpu.SemaphoreType.DMA((N_DEV - 1,)),
            pltpu.SemaphoreType.DMA((N_DEV - 1,)),
            pltpu.SemaphoreType.DMA((N_DEV - 1,)),
        ],
        compiler_params=pltpu.CompilerParams(collective_id=0),
    )(x, Wq, Wk, Wv, WqP, WkP, Wo, cos, sin)


# baseline (device time: 249486 ns/iter reference)
import os

import jax
import jax.numpy as jnp
from jax import lax
from jax.experimental import pallas as pl
from jax.experimental.pallas import tpu as pltpu

jax.config.update(
    "jax_compilation_cache_dir",
    os.path.join(os.path.dirname(os.path.abspath(__file__)), ".jaxcache"),
)
jax.config.update("jax_persistent_cache_min_compile_time_secs", 1.0)
jax.config.update("jax_persistent_cache_min_entry_size_bytes", 0)

N_DEV = 4
SQ = 1024
D = 1024
HQ = 8
DH = 128
SCALE = 0.08838834764831843


def _rot_cols(w):
    w2 = w.reshape(w.shape[0], -1, 2)
    return jnp.stack([-w2[..., 1], w2[..., 0]], axis=-1).reshape(w.shape)


def kernel(x, Wq, Wk, Wv, Wo):
    bf16 = jnp.bfloat16
    x = x.reshape(SQ, D).astype(bf16)
    Wq = Wq.astype(bf16)
    Wk = Wk.astype(bf16)
    Wv = Wv.astype(bf16)
    Wo = Wo.astype(bf16)
    WqP = _rot_cols(Wq)
    WkP = _rot_cols(Wk)

    inv = 10000.0 ** (-jnp.arange(0, DH, 2, dtype=jnp.float32) / DH)
    ang = jnp.arange(SQ, dtype=jnp.float32)[:, None] * inv[None, :]
    cos = jnp.tile(jnp.repeat(jnp.cos(ang), 2, axis=-1), (1, HQ)).astype(bf16)
    sin = jnp.tile(jnp.repeat(jnp.sin(ang), 2, axis=-1), (1, HQ)).astype(bf16)

    def body(x_ref, wq_ref, wk_ref, wv_ref, wqp_ref, wkp_ref, wo_ref,
             cos_ref, sin_ref, out_ref,
             xall, psend, acc, x_send_sems, x_recv_sems,
             p_send_sems, p_recv_sems):
        me = lax.axis_index("i")

        barrier = pltpu.get_barrier_semaphore()
        for r in range(1, N_DEV):
            pl.semaphore_signal(
                barrier, inc=1,
                device_id=((me + r) % N_DEV,),
                device_id_type=pl.DeviceIdType.MESH,
            )
        pl.semaphore_wait(barrier, N_DEV - 1)

        x_sends = []
        for r in range(1, N_DEV):
            rdma = pltpu.make_async_remote_copy(
                src_ref=x_ref,
                dst_ref=xall.at[3 - r],
                send_sem=x_send_sems.at[r - 1],
                recv_sem=x_recv_sems.at[3 - r],
                device_id=((me + r) % N_DEV,),
                device_id_type=pl.DeviceIdType.MESH,
            )
            rdma.start()
            x_sends.append(rdma)

        def mm(a, b):
            return lax.dot_general(
                a, b, (((1,), (0,)), ((), ())),
                preferred_element_type=jnp.float32,
            )

        def partial_for(xb):
            cosv = cos_ref[...].astype(jnp.float32)
            sinv = sin_ref[...].astype(jnp.float32)
            q = (mm(xb, wq_ref[...]) * cosv + mm(xb, wqp_ref[...]) * sinv).astype(bf16)
            k = (mm(xb, wk_ref[...]) * cosv + mm(xb, wkp_ref[...]) * sinv).astype(bf16)
            v = mm(xb, wv_ref[...]).astype(bf16)
            out = None
            for h in range(HQ):
                sl = slice(h * DH, (h + 1) * DH)
                s = lax.dot_general(
                    q[:, sl], k[:, sl], (((1,), (1,)), ((), ())),
                    preferred_element_type=jnp.float32,
                ) * SCALE
                m = jnp.max(s, axis=-1, keepdims=True)
                e = jnp.exp(s - m)
                den = jnp.sum(e, axis=-1, keepdims=True)
                ctx = (mm(e.astype(bf16), v[:, sl]) / den).astype(bf16)
                ph = mm(ctx, wo_ref[sl, :])
                out = ph if out is None else out + ph
            return out

        out_ref[0, :, :] = partial_for(x_ref[...])

        p_sends = []
        for s in range(N_DEV - 1):
            recv = pltpu.make_async_remote_copy(
                src_ref=x_ref,
                dst_ref=xall.at[s],
                send_sem=x_send_sems.at[0],
                recv_sem=x_recv_sems.at[s],
                device_id=(me,),
                device_id_type=pl.DeviceIdType.MESH,
            )
            recv.wait_recv()
            psend[s, :, :] = partial_for(xall[s]).astype(bf16)
            rdma = pltpu.make_async_remote_copy(
                src_ref=psend.at[s],
                dst_ref=acc.at[2 - s],
                send_sem=p_send_sems.at[s],
                recv_sem=p_recv_sems.at[2 - s],
                device_id=((me + s + 1) % N_DEV,),
                device_id_type=pl.DeviceIdType.MESH,
            )
            rdma.start()
            p_sends.append(rdma)

        for s in range(N_DEV - 1):
            recv = pltpu.make_async_remote_copy(
                src_ref=psend.at[0],
                dst_ref=acc.at[s],
                send_sem=p_send_sems.at[0],
                recv_sem=p_recv_sems.at[s],
                device_id=(me,),
                device_id_type=pl.DeviceIdType.MESH,
            )
            recv.wait_recv()

        out_ref[0, :, :] = out_ref[0, :, :] + (
            acc[0].astype(jnp.float32)
            + acc[1].astype(jnp.float32)
            + acc[2].astype(jnp.float32)
        )

        for rdma in x_sends + p_sends:
            rdma.wait_send()

    return pl.pallas_call(
        body,
        out_shape=jax.ShapeDtypeStruct((1, SQ, D), jnp.float32),
        in_specs=[pl.BlockSpec(memory_space=pltpu.VMEM)] * 9,
        out_specs=pl.BlockSpec(memory_space=pltpu.VMEM),
        scratch_shapes=[
            pltpu.VMEM((N_DEV - 1, SQ, D), jnp.bfloat16),
            pltpu.VMEM((N_DEV - 1, SQ, D), jnp.bfloat16),
            pltpu.VMEM((N_DEV - 1, SQ, D), jnp.bfloat16),
            pltpu.SemaphoreType.DMA((N_DEV - 1,)),
            pltpu.SemaphoreType.DMA((N_DEV - 1,)),
            pltpu.SemaphoreType.DMA((N_DEV - 1,)),
            pltpu.SemaphoreType.DMA((N_DEV - 1,)),
        ],
        compiler_params=pltpu.CompilerParams(
            collective_id=0, vmem_limit_bytes=100 * 1024 * 1024
        ),
    )(x, Wq, Wk, Wv, WqP, WkP, Wo, cos, sin)


# device time: 180218 ns/iter; 1.3844x vs baseline; 1.3844x over previous
import os

import jax
import jax.numpy as jnp
from jax import lax
from jax.experimental import pallas as pl
from jax.experimental.pallas import tpu as pltpu

jax.config.update(
    "jax_compilation_cache_dir",
    os.path.join(os.path.dirname(os.path.abspath(__file__)), ".jaxcache"),
)
jax.config.update("jax_persistent_cache_min_compile_time_secs", 1.0)
jax.config.update("jax_persistent_cache_min_entry_size_bytes", 0)

N_DEV = 4
SQ = 1024
D = 1024
HQ = 8
DH = 128
SCALE = 0.08838834764831843


def kernel(x, Wq, Wk, Wv, Wo):
    bf16 = jnp.bfloat16
    x = x.reshape(SQ, D).astype(bf16)
    Wq = Wq.astype(bf16)
    Wk = Wk.astype(bf16)
    Wv = Wv.astype(bf16)
    Wo = Wo.astype(bf16)

    inv = 10000.0 ** (-jnp.arange(0, DH, 2, dtype=jnp.float32) / DH)
    ang = jnp.arange(SQ, dtype=jnp.float32)[:, None] * inv[None, :]
    cos = jnp.tile(jnp.repeat(jnp.cos(ang), 2, axis=-1), (1, HQ)).astype(bf16)
    sin = jnp.tile(jnp.repeat(jnp.sin(ang), 2, axis=-1), (1, HQ)).astype(bf16)

    def body(x_ref, wq_ref, wk_ref, wv_ref, wo_ref, cos_ref, sin_ref,
             out_ref,
             qkv0, xall, psend, acc,
             x_send_sems, x_recv_sems, p_send_sems, p_recv_sems):
        me = lax.axis_index("i")

        barrier = pltpu.get_barrier_semaphore()
        for r in range(1, N_DEV):
            pl.semaphore_signal(
                barrier, inc=1,
                device_id=((me + r) % N_DEV,),
                device_id_type=pl.DeviceIdType.MESH,
            )
        pl.semaphore_wait(barrier, N_DEV - 1)

        x_sends = []
        for r in range(1, N_DEV):
            rdma = pltpu.make_async_remote_copy(
                src_ref=x_ref,
                dst_ref=xall.at[3 - r],
                send_sem=x_send_sems.at[r - 1],
                recv_sem=x_recv_sems.at[3 - r],
                device_id=((me + r) % N_DEV,),
                device_id_type=pl.DeviceIdType.MESH,
            )
            rdma.start()
            x_sends.append(rdma)

        def mm(a, b):
            return lax.dot_general(
                a, b, (((1,), (0,)), ((), ())),
                preferred_element_type=jnp.float32,
            )

        cosb = cos_ref[...]
        sinb = sin_ref[...]
        col_odd = lax.broadcasted_iota(jnp.int32, (SQ, D), 1) % 2 == 1

        def rope(t):
            tb = t.astype(bf16)
            rr = pltpu.roll(tb, 1, 1)
            rl = pltpu.roll(tb, D - 1, 1)
            tr = jnp.where(col_odd, rr, -rl)
            return tb * cosb + tr * sinb

        def qkv(xb):
            q = rope(mm(xb, wq_ref[...]))
            k = rope(mm(xb, wk_ref[...]))
            v = mm(xb, wv_ref[...]).astype(bf16)
            return q, k, v

        def attn(q, k, v):
            out = None
            for h in range(HQ):
                sl = slice(h * DH, (h + 1) * DH)
                s = lax.dot_general(
                    q[:, sl], k[:, sl], (((1,), (1,)), ((), ())),
                    preferred_element_type=jnp.float32,
                ) * SCALE
                e = jnp.exp(s)
                den = jnp.sum(e, axis=-1, keepdims=True)
                ctx = (mm(e.astype(bf16), v[:, sl]) / den).astype(bf16)
                ph = mm(ctx, wo_ref[sl, :])
                out = ph if out is None else out + ph
            return out

        q0, k0, v0 = qkv(x_ref[...])
        qkv0[0, :, :] = q0
        qkv0[1, :, :] = k0
        qkv0[2, :, :] = v0

        p_sends = []
        for s in (0, 2, 1):
            recv = pltpu.make_async_remote_copy(
                src_ref=x_ref,
                dst_ref=xall.at[s],
                send_sem=x_send_sems.at[0],
                recv_sem=x_recv_sems.at[s],
                device_id=(me,),
                device_id_type=pl.DeviceIdType.MESH,
            )
            recv.wait_recv()
            psend[s, :, :] = attn(*qkv(xall[s])).astype(bf16)
            rdma = pltpu.make_async_remote_copy(
                src_ref=psend.at[s],
                dst_ref=acc.at[2 - s],
                send_sem=p_send_sems.at[s],
                recv_sem=p_recv_sems.at[2 - s],
                device_id=((me + s + 1) % N_DEV,),
                device_id_type=pl.DeviceIdType.MESH,
            )
            rdma.start()
            p_sends.append(rdma)

        out_ref[0, :, :] = attn(qkv0[0], qkv0[1], qkv0[2])

        for s in range(N_DEV - 1):
            recv = pltpu.make_async_remote_copy(
                src_ref=psend.at[0],
                dst_ref=acc.at[s],
                send_sem=p_send_sems.at[0],
                recv_sem=p_recv_sems.at[s],
                device_id=(me,),
                device_id_type=pl.DeviceIdType.MESH,
            )
            recv.wait_recv()

        out_ref[0, :, :] = out_ref[0, :, :] + (
            acc[0].astype(jnp.float32)
            + acc[1].astype(jnp.float32)
            + acc[2].astype(jnp.float32)
        )

        for rdma in x_sends + p_sends:
            rdma.wait_send()

    return pl.pallas_call(
        body,
        out_shape=jax.ShapeDtypeStruct((1, SQ, D), jnp.float32),
        in_specs=[pl.BlockSpec(memory_space=pltpu.VMEM)] * 7,
        out_specs=pl.BlockSpec(memory_space=pltpu.VMEM),
        scratch_shapes=[
            pltpu.VMEM((3, SQ, D), jnp.bfloat16),
            pltpu.VMEM((N_DEV - 1, SQ, D), jnp.bfloat16),
            pltpu.VMEM((N_DEV - 1, SQ, D), jnp.bfloat16),
            pltpu.VMEM((N_DEV - 1, SQ, D), jnp.bfloat16),
            pltpu.SemaphoreType.DMA((N_DEV - 1,)),
            pltpu.SemaphoreType.DMA((N_DEV - 1,)),
            pltpu.SemaphoreType.DMA((N_DEV - 1,)),
            pltpu.SemaphoreType.DMA((N_DEV - 1,)),
        ],
        compiler_params=pltpu.CompilerParams(
            collective_id=0, vmem_limit_bytes=100 * 1024 * 1024
        ),
    )(x, Wq, Wk, Wv, Wo, cos, sin)


# device time: 170964 ns/iter; 1.4593x vs baseline; 1.0541x over previous
import os

import jax
import jax.numpy as jnp
from jax import lax
from jax.experimental import pallas as pl
from jax.experimental.pallas import tpu as pltpu

jax.config.update(
    "jax_compilation_cache_dir",
    os.path.join(os.path.dirname(os.path.abspath(__file__)), ".jaxcache"),
)
jax.config.update("jax_persistent_cache_min_compile_time_secs", 1.0)
jax.config.update("jax_persistent_cache_min_entry_size_bytes", 0)

N_DEV = 4
SQ = 1024
D = 1024
HQ = 8
DH = 128
SCALE = 0.08838834764831843


def kernel(x, Wq, Wk, Wv, Wo):
    bf16 = jnp.bfloat16
    x = x.reshape(SQ, D).astype(bf16)
    Wq = Wq.astype(bf16)
    Wk = Wk.astype(bf16)
    Wv = Wv.astype(bf16)
    Wo = Wo.astype(bf16)

    inv = 10000.0 ** (-jnp.arange(0, DH, 2, dtype=jnp.float32) / DH)
    ang = jnp.arange(SQ, dtype=jnp.float32)[:, None] * inv[None, :]
    cos = jnp.tile(jnp.repeat(jnp.cos(ang), 2, axis=-1), (1, HQ))
    sin = jnp.tile(jnp.repeat(jnp.sin(ang), 2, axis=-1), (1, HQ))
    cosb = cos.astype(bf16)
    sinb = sin.astype(bf16)
    cosq = (cos * SCALE).astype(bf16)
    sinq = (sin * SCALE).astype(bf16)

    def body(x_ref, wq_ref, wk_ref, wv_ref, wo_ref,
             cosb_ref, sinb_ref, cosq_ref, sinq_ref,
             out_ref,
             qkv0, xall, psend, acc,
             x_send_sems, x_recv_sems, p_send_sems, p_recv_sems):
        me = lax.axis_index("i")

        barrier = pltpu.get_barrier_semaphore()
        for r in range(1, N_DEV):
            pl.semaphore_signal(
                barrier, inc=1,
                device_id=((me + r) % N_DEV,),
                device_id_type=pl.DeviceIdType.MESH,
            )
        pl.semaphore_wait(barrier, N_DEV - 1)

        def x_send(r):
            rdma = pltpu.make_async_remote_copy(
                src_ref=x_ref,
                dst_ref=xall.at[3 - r],
                send_sem=x_send_sems.at[r - 1],
                recv_sem=x_recv_sems.at[3 - r],
                device_id=((me + r) % N_DEV,),
                device_id_type=pl.DeviceIdType.MESH,
            )
            rdma.start()
            return rdma

        x_sends = [x_send(1), x_send(3)]

        def mm(a, b):
            return lax.dot_general(
                a, b, (((1,), (0,)), ((), ())),
                preferred_element_type=jnp.float32,
            )

        col_odd = lax.broadcasted_iota(jnp.int32, (SQ, D), 1) % 2 == 1

        def rope(t, c, sn):
            tb = t.astype(bf16)
            rr = pltpu.roll(tb, 1, 1)
            rl = pltpu.roll(tb, D - 1, 1)
            tr = jnp.where(col_odd, rr, -rl)
            return tb * c + tr * sn

        def qkv(xb):
            q = rope(mm(xb, wq_ref[...]), cosq_ref[...], sinq_ref[...])
            k = rope(mm(xb, wk_ref[...]), cosb_ref[...], sinb_ref[...])
            v = mm(xb, wv_ref[...]).astype(bf16)
            return q, k, v

        def attn(q, k, v, h0, h1):
            out = None
            for h in range(h0, h1):
                sl = slice(h * DH, (h + 1) * DH)
                s = lax.dot_general(
                    q[:, sl], k[:, sl], (((1,), (1,)), ((), ())),
                    preferred_element_type=jnp.float32,
                )
                e = jnp.exp(s).astype(bf16)
                den = jnp.sum(e.astype(jnp.float32), axis=-1, keepdims=True)
                ctx = (mm(e, v[:, sl]) / den).astype(bf16)
                ph = mm(ctx, wo_ref[sl, :])
                out = ph if out is None else out + ph
            return out

        q0, k0, v0 = qkv(x_ref[...])
        qkv0[0, :, :] = q0
        qkv0[1, :, :] = k0
        qkv0[2, :, :] = v0
        x_sends.append(x_send(2))

        OWN_SPLIT = 2
        out_ref[0, :, :] = attn(qkv0[0], qkv0[1], qkv0[2], 0, OWN_SPLIT)

        p_sends = []
        for s in (0, 2, 1):
            recv = pltpu.make_async_remote_copy(
                src_ref=x_ref,
                dst_ref=xall.at[s],
                send_sem=x_send_sems.at[0],
                recv_sem=x_recv_sems.at[s],
                device_id=(me,),
                device_id_type=pl.DeviceIdType.MESH,
            )
            recv.wait_recv()
            qs, ks, vs = qkv(xall[s])
            psend[s, :, :] = attn(qs, ks, vs, 0, HQ).astype(bf16)
            rdma = pltpu.make_async_remote_copy(
                src_ref=psend.at[s],
                dst_ref=acc.at[2 - s],
                send_sem=p_send_sems.at[s],
                recv_sem=p_recv_sems.at[2 - s],
                device_id=((me + s + 1) % N_DEV,),
                device_id_type=pl.DeviceIdType.MESH,
            )
            rdma.start()
            p_sends.append(rdma)

        out_ref[0, :, :] = out_ref[0, :, :] + attn(
            qkv0[0], qkv0[1], qkv0[2], OWN_SPLIT, HQ
        )

        for s in range(N_DEV - 1):
            recv = pltpu.make_async_remote_copy(
                src_ref=psend.at[0],
                dst_ref=acc.at[s],
                send_sem=p_send_sems.at[0],
                recv_sem=p_recv_sems.at[s],
                device_id=(me,),
                device_id_type=pl.DeviceIdType.MESH,
            )
            recv.wait_recv()

        out_ref[0, :, :] = out_ref[0, :, :] + (
            acc[0].astype(jnp.float32)
            + acc[1].astype(jnp.float32)
            + acc[2].astype(jnp.float32)
        )

        for rdma in x_sends + p_sends:
            rdma.wait_send()

    return pl.pallas_call(
        body,
        out_shape=jax.ShapeDtypeStruct((1, SQ, D), jnp.float32),
        in_specs=[pl.BlockSpec(memory_space=pltpu.VMEM)] * 9,
        out_specs=pl.BlockSpec(memory_space=pltpu.VMEM),
        scratch_shapes=[
            pltpu.VMEM((3, SQ, D), jnp.bfloat16),
            pltpu.VMEM((N_DEV - 1, SQ, D), jnp.bfloat16),
            pltpu.VMEM((N_DEV - 1, SQ, D), jnp.bfloat16),
            pltpu.VMEM((N_DEV - 1, SQ, D), jnp.bfloat16),
            pltpu.SemaphoreType.DMA((N_DEV - 1,)),
            pltpu.SemaphoreType.DMA((N_DEV - 1,)),
            pltpu.SemaphoreType.DMA((N_DEV - 1,)),
            pltpu.SemaphoreType.DMA((N_DEV - 1,)),
        ],
        compiler_params=pltpu.CompilerParams(
            collective_id=0, vmem_limit_bytes=100 * 1024 * 1024
        ),
    )(x, Wq, Wk, Wv, Wo, cosb, sinb, cosq, sinq)


# device time: 161085 ns/iter; 1.5488x vs baseline; 1.0613x over previous
import os

import jax
import jax.numpy as jnp
import numpy as np
from jax import lax
from jax.experimental import pallas as pl
from jax.experimental.pallas import tpu as pltpu

jax.config.update(
    "jax_compilation_cache_dir",
    os.path.join(os.path.dirname(os.path.abspath(__file__)), ".jaxcache"),
)
jax.config.update("jax_persistent_cache_min_compile_time_secs", 1.0)
jax.config.update("jax_persistent_cache_min_entry_size_bytes", 0)

N_DEV = 4
SQ = 1024
D = 1024
HQ = 8
DH = 128
SCALE = 0.08838834764831843


def kernel(x, Wq, Wk, Wv, Wo):
    bf16 = jnp.bfloat16
    x = x.reshape(SQ, D).astype(bf16)
    Wqkv = jnp.concatenate([Wq, Wk, Wv], axis=1).astype(bf16)
    Wo = Wo.astype(bf16)

    inv = 10000.0 ** (-np.arange(0, DH, 2, dtype=np.float32) / DH)
    ang = np.arange(SQ, dtype=np.float32)[:, None] * inv[None, :]
    cos = np.tile(np.repeat(np.cos(ang), 2, axis=-1), (1, HQ))
    sin = np.tile(np.repeat(np.sin(ang), 2, axis=-1), (1, HQ))
    cosb = jnp.asarray(cos, dtype=bf16)
    sinb = jnp.asarray(sin, dtype=bf16)
    cosq = jnp.asarray(cos * SCALE, dtype=bf16)
    sinq = jnp.asarray(sin * SCALE, dtype=bf16)

    def body(x_ref, wqkv_ref, wo_ref,
             cosb_ref, sinb_ref, cosq_ref, sinq_ref,
             out_ref,
             qkv0, xall, psend, acc,
             x_send_sems, x_recv_sems, p_send_sems, p_recv_sems):
        me = lax.axis_index("i")

        barrier = pltpu.get_barrier_semaphore()
        for r in range(1, N_DEV):
            pl.semaphore_signal(
                barrier, inc=1,
                device_id=((me + r) % N_DEV,),
                device_id_type=pl.DeviceIdType.MESH,
            )
        pl.semaphore_wait(barrier, N_DEV - 1)

        def x_send(r):
            rdma = pltpu.make_async_remote_copy(
                src_ref=x_ref,
                dst_ref=xall.at[3 - r],
                send_sem=x_send_sems.at[r - 1],
                recv_sem=x_recv_sems.at[3 - r],
                device_id=((me + r) % N_DEV,),
                device_id_type=pl.DeviceIdType.MESH,
            )
            rdma.start()
            return rdma

        x_sends = [x_send(1), x_send(3)]

        def mm(a, b):
            return lax.dot_general(
                a, b, (((1,), (0,)), ((), ())),
                preferred_element_type=jnp.float32,
            )

        col_odd = lax.broadcasted_iota(jnp.int32, (SQ, D), 1) % 2 == 1

        def rope(t, c, sn):
            tb = t.astype(bf16)
            rr = pltpu.roll(tb, 1, 1)
            rl = pltpu.roll(tb, D - 1, 1)
            tr = jnp.where(col_odd, rr, -rl)
            return tb * c + tr * sn

        def qkv(xb):
            big = mm(xb, wqkv_ref[...])
            q = rope(big[:, :D], cosq_ref[...], sinq_ref[...])
            k = rope(big[:, D:2 * D], cosb_ref[...], sinb_ref[...])
            v = big[:, 2 * D:].astype(bf16)
            return q, k, v

        def attn(q, k, v, h0, h1):
            out = None
            for h in range(h0, h1):
                sl = slice(h * DH, (h + 1) * DH)
                s = lax.dot_general(
                    q[:, sl], k[:, sl], (((1,), (1,)), ((), ())),
                    preferred_element_type=jnp.float32,
                )
                e = jnp.exp(s).astype(bf16)
                den = jnp.sum(e.astype(jnp.float32), axis=-1, keepdims=True)
                ctx = (mm(e, v[:, sl]) / den).astype(bf16)
                ph = mm(ctx, wo_ref[sl, :])
                out = ph if out is None else out + ph
            return out

        q0, k0, v0 = qkv(x_ref[...])
        qkv0[0, :, :] = q0
        qkv0[1, :, :] = k0
        qkv0[2, :, :] = v0
        x_sends.append(x_send(2))

        OWN_SPLIT = 2
        out_ref[0, :, :] = attn(qkv0[0], qkv0[1], qkv0[2], 0, OWN_SPLIT).astype(bf16)

        p_sends = []
        for s in (0, 2, 1):
            recv = pltpu.make_async_remote_copy(
                src_ref=x_ref,
                dst_ref=xall.at[s],
                send_sem=x_send_sems.at[0],
                recv_sem=x_recv_sems.at[s],
                device_id=(me,),
                device_id_type=pl.DeviceIdType.MESH,
            )
            recv.wait_recv()
            qs, ks, vs = qkv(xall[s])
            psend[s, :, :] = attn(qs, ks, vs, 0, HQ).astype(bf16)
            rdma = pltpu.make_async_remote_copy(
                src_ref=psend.at[s],
                dst_ref=acc.at[2 - s],
                send_sem=p_send_sems.at[s],
                recv_sem=p_recv_sems.at[2 - s],
                device_id=((me + s + 1) % N_DEV,),
                device_id_type=pl.DeviceIdType.MESH,
            )
            rdma.start()
            p_sends.append(rdma)

        rest = attn(qkv0[0], qkv0[1], qkv0[2], OWN_SPLIT, HQ)

        for s in range(N_DEV - 1):
            recv = pltpu.make_async_remote_copy(
                src_ref=psend.at[0],
                dst_ref=acc.at[s],
                send_sem=p_send_sems.at[0],
                recv_sem=p_recv_sems.at[s],
                device_id=(me,),
                device_id_type=pl.DeviceIdType.MESH,
            )
            recv.wait_recv()

        out_ref[0, :, :] = (
            out_ref[0, :, :].astype(jnp.float32)
            + rest
            + acc[0].astype(jnp.float32)
            + acc[1].astype(jnp.float32)
            + acc[2].astype(jnp.float32)
        ).astype(bf16)

        for rdma in x_sends + p_sends:
            rdma.wait_send()

    return pl.pallas_call(
        body,
        out_shape=jax.ShapeDtypeStruct((1, SQ, D), jnp.bfloat16),
        in_specs=[pl.BlockSpec(memory_space=pltpu.VMEM)] * 7,
        out_specs=pl.BlockSpec(memory_space=pltpu.VMEM),
        scratch_shapes=[
            pltpu.VMEM((3, SQ, D), jnp.bfloat16),
            pltpu.VMEM((N_DEV - 1, SQ, D), jnp.bfloat16),
            pltpu.VMEM((N_DEV - 1, SQ, D), jnp.bfloat16),
            pltpu.VMEM((N_DEV - 1, SQ, D), jnp.bfloat16),
            pltpu.SemaphoreType.DMA((N_DEV - 1,)),
            pltpu.SemaphoreType.DMA((N_DEV - 1,)),
            pltpu.SemaphoreType.DMA((N_DEV - 1,)),
            pltpu.SemaphoreType.DMA((N_DEV - 1,)),
        ],
        compiler_params=pltpu.CompilerParams(
            collective_id=0, vmem_limit_bytes=100 * 1024 * 1024
        ),
    )(x, Wqkv, Wo, cosb, sinb, cosq, sinq)


# device time: 158479 ns/iter; 1.5743x vs baseline; 1.0164x over previous
import os

import jax
import jax.numpy as jnp
import numpy as np
from jax import lax
from jax.experimental import pallas as pl
from jax.experimental.pallas import tpu as pltpu

jax.config.update(
    "jax_compilation_cache_dir",
    os.path.join(os.path.dirname(os.path.abspath(__file__)), ".jaxcache"),
)
jax.config.update("jax_persistent_cache_min_compile_time_secs", 1.0)
jax.config.update("jax_persistent_cache_min_entry_size_bytes", 0)

N_DEV = 4
SQ = 1024
D = 1024
HQ = 8
DH = 128
SCALE = 0.08838834764831843


def kernel(x, Wq, Wk, Wv, Wo):
    bf16 = jnp.bfloat16
    x = x.reshape(SQ, D).astype(bf16)
    Wqkv = jnp.concatenate([Wq, Wk, Wv], axis=1).astype(bf16)
    Wo = Wo.astype(bf16)

    inv = 10000.0 ** (-np.arange(0, DH, 2, dtype=np.float32) / DH)
    ang = np.arange(SQ, dtype=np.float32)[:, None] * inv[None, :]
    cos = np.tile(np.repeat(np.cos(ang), 2, axis=-1), (1, HQ))
    sin = np.tile(np.repeat(np.sin(ang), 2, axis=-1), (1, HQ))
    cosb = jnp.asarray(cos, dtype=bf16)
    sinb = jnp.asarray(sin, dtype=bf16)
    cosq = jnp.asarray(cos * SCALE, dtype=bf16)
    sinq = jnp.asarray(sin * SCALE, dtype=bf16)

    def body(x_ref, wqkv_ref, wo_ref,
             cosb_ref, sinb_ref, cosq_ref, sinq_ref,
             out_ref,
             qkv0, xall, psend, acc,
             x_send_sems, x_recv_sems, p_send_sems, p_recv_sems):
        me = lax.axis_index("i")

        barrier = pltpu.get_barrier_semaphore()
        for r in range(1, N_DEV):
            pl.semaphore_signal(
                barrier, inc=1,
                device_id=((me + r) % N_DEV,),
                device_id_type=pl.DeviceIdType.MESH,
            )
        pl.semaphore_wait(barrier, N_DEV - 1)

        def x_send(r):
            rdma = pltpu.make_async_remote_copy(
                src_ref=x_ref,
                dst_ref=xall.at[3 - r],
                send_sem=x_send_sems.at[r - 1],
                recv_sem=x_recv_sems.at[3 - r],
                device_id=((me + r) % N_DEV,),
                device_id_type=pl.DeviceIdType.MESH,
            )
            rdma.start()
            return rdma

        x_sends = [x_send(1), x_send(3)]

        def mm(a, b):
            return lax.dot_general(
                a, b, (((1,), (0,)), ((), ())),
                preferred_element_type=jnp.float32,
            )

        col_odd = lax.broadcasted_iota(jnp.int32, (SQ, D), 1) % 2 == 1

        def rope(t, c, sn):
            tb = t.astype(bf16)
            rr = pltpu.roll(tb, 1, 1)
            rl = pltpu.roll(tb, D - 1, 1)
            tr = jnp.where(col_odd, rr, -rl)
            return tb * c + tr * sn

        def qkv(xb):
            big = mm(xb, wqkv_ref[...])
            q = rope(big[:, :D], cosq_ref[...], sinq_ref[...])
            k = rope(big[:, D:2 * D], cosb_ref[...], sinb_ref[...])
            v = big[:, 2 * D:].astype(bf16)
            return q, k, v

        ones_col = jnp.ones((SQ, DH), bf16)

        def attn(q, k, v, h0, h1):
            out = None
            for h in range(h0, h1):
                sl = slice(h * DH, (h + 1) * DH)
                s = lax.dot_general(
                    q[:, sl], k[:, sl], (((1,), (1,)), ((), ())),
                    preferred_element_type=jnp.float32,
                )
                e = jnp.exp(s).astype(bf16)
                ctxa = mm(e, jnp.concatenate([v[:, sl], ones_col], axis=1))
                ctx = (ctxa[:, :DH] / ctxa[:, DH:DH + 1]).astype(bf16)
                ph = mm(ctx, wo_ref[sl, :])
                out = ph if out is None else out + ph
            return out

        q0, k0, v0 = qkv(x_ref[...])
        qkv0[0, :, :] = q0
        qkv0[1, :, :] = k0
        qkv0[2, :, :] = v0
        x_sends.append(x_send(2))

        OWN_SPLIT = 2
        out_ref[0, :, :] = attn(qkv0[0], qkv0[1], qkv0[2], 0, OWN_SPLIT).astype(bf16)

        p_sends = []
        for s in (0, 2, 1):
            recv = pltpu.make_async_remote_copy(
                src_ref=x_ref,
                dst_ref=xall.at[s],
                send_sem=x_send_sems.at[0],
                recv_sem=x_recv_sems.at[s],
                device_id=(me,),
                device_id_type=pl.DeviceIdType.MESH,
            )
            recv.wait_recv()
            qs, ks, vs = qkv(xall[s])
            psend[s, :, :] = attn(qs, ks, vs, 0, HQ).astype(bf16)
            rdma = pltpu.make_async_remote_copy(
                src_ref=psend.at[s],
                dst_ref=acc.at[2 - s],
                send_sem=p_send_sems.at[s],
                recv_sem=p_recv_sems.at[2 - s],
                device_id=((me + s + 1) % N_DEV,),
                device_id_type=pl.DeviceIdType.MESH,
            )
            rdma.start()
            p_sends.append(rdma)

        rest = attn(qkv0[0], qkv0[1], qkv0[2], OWN_SPLIT, HQ)

        for s in range(N_DEV - 1):
            recv = pltpu.make_async_remote_copy(
                src_ref=psend.at[0],
                dst_ref=acc.at[s],
                send_sem=p_send_sems.at[0],
                recv_sem=p_recv_sems.at[s],
                device_id=(me,),
                device_id_type=pl.DeviceIdType.MESH,
            )
            recv.wait_recv()

        out_ref[0, :, :] = (
            out_ref[0, :, :].astype(jnp.float32)
            + rest
            + acc[0].astype(jnp.float32)
            + acc[1].astype(jnp.float32)
            + acc[2].astype(jnp.float32)
        ).astype(bf16)

        for rdma in x_sends + p_sends:
            rdma.wait_send()

    return pl.pallas_call(
        body,
        out_shape=jax.ShapeDtypeStruct((1, SQ, D), jnp.bfloat16),
        in_specs=[pl.BlockSpec(memory_space=pltpu.VMEM)] * 7,
        out_specs=pl.BlockSpec(memory_space=pltpu.VMEM),
        scratch_shapes=[
            pltpu.VMEM((3, SQ, D), jnp.bfloat16),
            pltpu.VMEM((N_DEV - 1, SQ, D), jnp.bfloat16),
            pltpu.VMEM((N_DEV - 1, SQ, D), jnp.bfloat16),
            pltpu.VMEM((N_DEV - 1, SQ, D), jnp.bfloat16),
            pltpu.SemaphoreType.DMA((N_DEV - 1,)),
            pltpu.SemaphoreType.DMA((N_DEV - 1,)),
            pltpu.SemaphoreType.DMA((N_DEV - 1,)),
            pltpu.SemaphoreType.DMA((N_DEV - 1,)),
        ],
        compiler_params=pltpu.CompilerParams(
            collective_id=0, vmem_limit_bytes=100 * 1024 * 1024
        ),
    )(x, Wqkv, Wo, cosb, sinb, cosq, sinq)


# device time: 153352 ns/iter; 1.6269x vs baseline; 1.0334x over previous
import os

import jax
import jax.numpy as jnp
import numpy as np
from jax import lax
from jax.experimental import pallas as pl
from jax.experimental.pallas import tpu as pltpu

jax.config.update(
    "jax_compilation_cache_dir",
    os.path.join(os.path.dirname(os.path.abspath(__file__)), ".jaxcache"),
)
jax.config.update("jax_persistent_cache_min_compile_time_secs", 1.0)
jax.config.update("jax_persistent_cache_min_entry_size_bytes", 0)

N_DEV = 4
SQ = 1024
D = 1024
HQ = 8
DH = 128
SCALE = 0.08838834764831843


def kernel(x, Wq, Wk, Wv, Wo):
    bf16 = jnp.bfloat16
    x = x.reshape(SQ, D).astype(bf16)
    Wqkv = jnp.concatenate([Wq, Wk, Wv], axis=1).astype(bf16)
    Wo = Wo.astype(bf16)

    inv = 10000.0 ** (-np.arange(0, DH, 2, dtype=np.float32) / DH)
    ang = np.arange(SQ, dtype=np.float32)[:, None] * inv[None, :]
    cos = np.tile(np.repeat(np.cos(ang), 2, axis=-1), (1, HQ))
    sin = np.tile(np.repeat(np.sin(ang), 2, axis=-1), (1, HQ))
    cosb = jnp.asarray(cos, dtype=bf16)
    sinb = jnp.asarray(sin, dtype=bf16)
    cosq = jnp.asarray(cos * SCALE, dtype=bf16)
    sinq = jnp.asarray(sin * SCALE, dtype=bf16)

    def body(x_ref, wqkv_ref, wo_ref,
             cosb_ref, sinb_ref, cosq_ref, sinq_ref,
             out_ref,
             qkv0, xall, psend, acc,
             x_send_sems, x_recv_sems, p_send_sems, p_recv_sems):
        me = lax.axis_index("i")

        barrier = pltpu.get_barrier_semaphore()
        for r in range(1, N_DEV):
            pl.semaphore_signal(
                barrier, inc=1,
                device_id=((me + r) % N_DEV,),
                device_id_type=pl.DeviceIdType.MESH,
            )
        pl.semaphore_wait(barrier, N_DEV - 1)

        def x_send(r):
            rdma = pltpu.make_async_remote_copy(
                src_ref=x_ref,
                dst_ref=xall.at[3 - r],
                send_sem=x_send_sems.at[r - 1],
                recv_sem=x_recv_sems.at[3 - r],
                device_id=((me + r) % N_DEV,),
                device_id_type=pl.DeviceIdType.MESH,
            )
            rdma.start()
            return rdma

        x_sends = [x_send(1), x_send(3)]

        def mm(a, b):
            return lax.dot_general(
                a, b, (((1,), (0,)), ((), ())),
                preferred_element_type=jnp.float32,
            )

        col_odd = lax.broadcasted_iota(jnp.int32, (SQ, D), 1) % 2 == 1

        def rope(t, c, sn):
            tb = t.astype(bf16)
            rr = pltpu.roll(tb, 1, 1)
            rl = pltpu.roll(tb, D - 1, 1)
            tr = jnp.where(col_odd, rr, -rl)
            return tb * c + tr * sn

        def qkv(xb):
            big = mm(xb, wqkv_ref[...])
            q = rope(big[:, :D], cosq_ref[...], sinq_ref[...])
            k = rope(big[:, D:2 * D], cosb_ref[...], sinb_ref[...])
            v = big[:, 2 * D:].astype(bf16)
            return q, k, v

        ones_col = jnp.ones((SQ, DH), bf16)

        def attn(q, k, v, h0, h1, r0=0, r1=SQ):
            out = None
            for h in range(h0, h1):
                sl = slice(h * DH, (h + 1) * DH)
                s = lax.dot_general(
                    q[r0:r1, sl], k[:, sl], (((1,), (1,)), ((), ())),
                    preferred_element_type=jnp.float32,
                )
                e = jnp.exp(s).astype(bf16)
                ctxa = mm(e, jnp.concatenate([v[:, sl], ones_col], axis=1))
                ctx = (ctxa[:, :DH] / ctxa[:, DH:DH + 1]).astype(bf16)
                ph = mm(ctx, wo_ref[sl, :])
                out = ph if out is None else out + ph
            return out

        q0, k0, v0 = qkv(x_ref[...])
        qkv0[0, :, :] = q0
        qkv0[1, :, :] = k0
        qkv0[2, :, :] = v0
        x_sends.append(x_send(2))

        OWN_SPLIT = 2
        out_ref[0, :, :] = attn(qkv0[0], qkv0[1], qkv0[2], 0, OWN_SPLIT).astype(bf16)

        p_sends = []
        for s in (0, 2, 1):
            recv = pltpu.make_async_remote_copy(
                src_ref=x_ref,
                dst_ref=xall.at[s],
                send_sem=x_send_sems.at[0],
                recv_sem=x_recv_sems.at[s],
                device_id=(me,),
                device_id_type=pl.DeviceIdType.MESH,
            )
            recv.wait_recv()
            qs, ks, vs = qkv(xall[s])
            for c in range(2):
                r0, r1 = c * (SQ // 2), (c + 1) * (SQ // 2)
                psend[s, r0:r1, :] = attn(qs, ks, vs, 0, HQ, r0, r1).astype(bf16)
                rdma = pltpu.make_async_remote_copy(
                    src_ref=psend.at[s, r0:r1],
                    dst_ref=acc.at[2 - s, r0:r1],
                    send_sem=p_send_sems.at[s, c],
                    recv_sem=p_recv_sems.at[2 - s, c],
                    device_id=((me + s + 1) % N_DEV,),
                    device_id_type=pl.DeviceIdType.MESH,
                )
                rdma.start()
                p_sends.append(rdma)

        rest = attn(qkv0[0], qkv0[1], qkv0[2], OWN_SPLIT, HQ)

        for s in range(N_DEV - 1):
            for c in range(2):
                r0, r1 = c * (SQ // 2), (c + 1) * (SQ // 2)
                recv = pltpu.make_async_remote_copy(
                    src_ref=psend.at[0, r0:r1],
                    dst_ref=acc.at[s, r0:r1],
                    send_sem=p_send_sems.at[0, 0],
                    recv_sem=p_recv_sems.at[s, c],
                    device_id=(me,),
                    device_id_type=pl.DeviceIdType.MESH,
                )
                recv.wait_recv()

        out_ref[0, :, :] = (
            out_ref[0, :, :].astype(jnp.float32)
            + rest
            + acc[0].astype(jnp.float32)
            + acc[1].astype(jnp.float32)
            + acc[2].astype(jnp.float32)
        ).astype(bf16)

        for rdma in x_sends + p_sends:
            rdma.wait_send()

    return pl.pallas_call(
        body,
        out_shape=jax.ShapeDtypeStruct((1, SQ, D), jnp.bfloat16),
        in_specs=[pl.BlockSpec(memory_space=pltpu.VMEM)] * 7,
        out_specs=pl.BlockSpec(memory_space=pltpu.VMEM),
        scratch_shapes=[
            pltpu.VMEM((3, SQ, D), jnp.bfloat16),
            pltpu.VMEM((N_DEV - 1, SQ, D), jnp.bfloat16),
            pltpu.VMEM((N_DEV - 1, SQ, D), jnp.bfloat16),
            pltpu.VMEM((N_DEV - 1, SQ, D), jnp.bfloat16),
            pltpu.SemaphoreType.DMA((N_DEV - 1,)),
            pltpu.SemaphoreType.DMA((N_DEV - 1,)),
            pltpu.SemaphoreType.DMA((N_DEV - 1, 2)),
            pltpu.SemaphoreType.DMA((N_DEV - 1, 2)),
        ],
        compiler_params=pltpu.CompilerParams(
            collective_id=0, vmem_limit_bytes=100 * 1024 * 1024
        ),
    )(x, Wqkv, Wo, cosb, sinb, cosq, sinq)


# device time: 152836 ns/iter; 1.6324x vs baseline; 1.0034x over previous
import os

import jax
import jax.numpy as jnp
import numpy as np
from jax import lax
from jax.experimental import pallas as pl
from jax.experimental.pallas import tpu as pltpu

jax.config.update(
    "jax_compilation_cache_dir",
    os.path.join(os.path.dirname(os.path.abspath(__file__)), ".jaxcache"),
)
jax.config.update("jax_persistent_cache_min_compile_time_secs", 1.0)
jax.config.update("jax_persistent_cache_min_entry_size_bytes", 0)

N_DEV = 4
SQ = 1024
D = 1024
HQ = 8
DH = 128
SCALE = 0.08838834764831843


def kernel(x, Wq, Wk, Wv, Wo):
    bf16 = jnp.bfloat16
    x = x.reshape(SQ, D).astype(bf16)
    Wqkv = jnp.concatenate([Wq, Wk, Wv], axis=1).astype(bf16)
    Wo = Wo.astype(bf16)

    inv = 10000.0 ** (-np.arange(0, DH, 2, dtype=np.float32) / DH)
    ang = np.arange(SQ, dtype=np.float32)[:, None] * inv[None, :]
    cos = np.tile(np.repeat(np.cos(ang), 2, axis=-1), (1, HQ))
    sin = np.tile(np.repeat(np.sin(ang), 2, axis=-1), (1, HQ))
    cosb = jnp.asarray(cos, dtype=bf16)
    sinb = jnp.asarray(sin, dtype=bf16)
    cosq = jnp.asarray(cos * SCALE, dtype=bf16)
    sinq = jnp.asarray(sin * SCALE, dtype=bf16)

    def body(x_ref, wqkv_ref, wo_ref,
             cosb_ref, sinb_ref, cosq_ref, sinq_ref,
             out_ref,
             qkv0, xall, psend, acc,
             x_send_sems, x_recv_sems, p_send_sems, p_recv_sems):
        me = lax.axis_index("i")

        barrier = pltpu.get_barrier_semaphore()
        for r in range(1, N_DEV):
            pl.semaphore_signal(
                barrier, inc=1,
                device_id=((me + r) % N_DEV,),
                device_id_type=pl.DeviceIdType.MESH,
            )
        pl.semaphore_wait(barrier, N_DEV - 1)

        def x_send(r):
            rdma = pltpu.make_async_remote_copy(
                src_ref=x_ref,
                dst_ref=xall.at[3 - r],
                send_sem=x_send_sems.at[r - 1],
                recv_sem=x_recv_sems.at[3 - r],
                device_id=((me + r) % N_DEV,),
                device_id_type=pl.DeviceIdType.MESH,
            )
            rdma.start()
            return rdma

        x_sends = [x_send(1), x_send(3)]

        def mm(a, b):
            return lax.dot_general(
                a, b, (((1,), (0,)), ((), ())),
                preferred_element_type=jnp.float32,
            )

        col_odd = lax.broadcasted_iota(jnp.int32, (SQ, D), 1) % 2 == 1

        def rope(t, c, sn):
            tb = t.astype(bf16)
            rr = pltpu.roll(tb, 1, 1)
            rl = pltpu.roll(tb, D - 1, 1)
            tr = jnp.where(col_odd, rr, -rl)
            return tb * c + tr * sn

        def qkv(xb):
            big = mm(xb, wqkv_ref[...])
            q = rope(big[:, :D], cosq_ref[...], sinq_ref[...])
            k = rope(big[:, D:2 * D], cosb_ref[...], sinb_ref[...])
            v = big[:, 2 * D:].astype(bf16)
            return q, k, v

        ones_col = jnp.ones((SQ, DH), bf16)

        def attn(q, k, v, h0, h1, r0=0, r1=SQ):
            out = None
            for h in range(h0, h1):
                sl = slice(h * DH, (h + 1) * DH)
                s = lax.dot_general(
                    q[r0:r1, sl], k[:, sl], (((1,), (1,)), ((), ())),
                    preferred_element_type=jnp.float32,
                )
                e = jnp.exp(s).astype(bf16)
                ctxa = mm(e, jnp.concatenate([v[:, sl], ones_col], axis=1))
                ctx = (ctxa[:, :DH] / ctxa[:, DH:DH + 1]).astype(bf16)
                ph = mm(ctx, wo_ref[sl, :])
                out = ph if out is None else out + ph
            return out

        q0, k0, v0 = qkv(x_ref[...])
        qkv0[0, :, :] = q0
        qkv0[1, :, :] = k0
        qkv0[2, :, :] = v0
        x_sends.append(x_send(2))

        OWN_SPLIT = 2
        out_ref[0, :, :] = attn(qkv0[0], qkv0[1], qkv0[2], 0, OWN_SPLIT).astype(bf16)

        p_sends = []
        for s in (0, 1, 2):
            recv = pltpu.make_async_remote_copy(
                src_ref=x_ref,
                dst_ref=xall.at[s],
                send_sem=x_send_sems.at[0],
                recv_sem=x_recv_sems.at[s],
                device_id=(me,),
                device_id_type=pl.DeviceIdType.MESH,
            )
            recv.wait_recv()
            qs, ks, vs = qkv(xall[s])
            for c in range(2):
                r0, r1 = c * (SQ // 2), (c + 1) * (SQ // 2)
                psend[s, r0:r1, :] = attn(qs, ks, vs, 0, HQ, r0, r1).astype(bf16)
                rdma = pltpu.make_async_remote_copy(
                    src_ref=psend.at[s, r0:r1],
                    dst_ref=acc.at[2 - s, r0:r1],
                    send_sem=p_send_sems.at[s, c],
                    recv_sem=p_recv_sems.at[2 - s, c],
                    device_id=((me + s + 1) % N_DEV,),
                    device_id_type=pl.DeviceIdType.MESH,
                )
                rdma.start()
                p_sends.append(rdma)

        rest = attn(qkv0[0], qkv0[1], qkv0[2], OWN_SPLIT, HQ)

        for s in range(N_DEV - 1):
            for c in range(2):
                r0, r1 = c * (SQ // 2), (c + 1) * (SQ // 2)
                recv = pltpu.make_async_remote_copy(
                    src_ref=psend.at[0, r0:r1],
                    dst_ref=acc.at[s, r0:r1],
                    send_sem=p_send_sems.at[0, 0],
                    recv_sem=p_recv_sems.at[s, c],
                    device_id=(me,),
                    device_id_type=pl.DeviceIdType.MESH,
                )
                recv.wait_recv()

        out_ref[0, :, :] = (
            out_ref[0, :, :].astype(jnp.float32)
            + rest
            + acc[0].astype(jnp.float32)
            + acc[1].astype(jnp.float32)
            + acc[2].astype(jnp.float32)
        ).astype(bf16)

        for rdma in x_sends + p_sends:
            rdma.wait_send()

    return pl.pallas_call(
        body,
        out_shape=jax.ShapeDtypeStruct((1, SQ, D), jnp.bfloat16),
        in_specs=[pl.BlockSpec(memory_space=pltpu.VMEM)] * 7,
        out_specs=pl.BlockSpec(memory_space=pltpu.VMEM),
        scratch_shapes=[
            pltpu.VMEM((3, SQ, D), jnp.bfloat16),
            pltpu.VMEM((N_DEV - 1, SQ, D), jnp.bfloat16),
            pltpu.VMEM((N_DEV - 1, SQ, D), jnp.bfloat16),
            pltpu.VMEM((N_DEV - 1, SQ, D), jnp.bfloat16),
            pltpu.SemaphoreType.DMA((N_DEV - 1,)),
            pltpu.SemaphoreType.DMA((N_DEV - 1,)),
            pltpu.SemaphoreType.DMA((N_DEV - 1, 2)),
            pltpu.SemaphoreType.DMA((N_DEV - 1, 2)),
        ],
        compiler_params=pltpu.CompilerParams(
            collective_id=0, vmem_limit_bytes=100 * 1024 * 1024
        ),
    )(x, Wqkv, Wo, cosb, sinb, cosq, sinq)


# device time: 145344 ns/iter; 1.7165x vs baseline; 1.0515x over previous
import os

import jax
import jax.numpy as jnp
import numpy as np
from jax import lax
from jax.experimental import pallas as pl
from jax.experimental.pallas import tpu as pltpu

jax.config.update(
    "jax_compilation_cache_dir",
    os.path.join(os.path.dirname(os.path.abspath(__file__)), ".jaxcache"),
)
jax.config.update("jax_persistent_cache_min_compile_time_secs", 1.0)
jax.config.update("jax_persistent_cache_min_entry_size_bytes", 0)

N_DEV = 4
SQ = 1024
D = 1024
HQ = 8
DH = 128
SCALE = 0.08838834764831843


def kernel(x, Wq, Wk, Wv, Wo):
    bf16 = jnp.bfloat16
    x = x.reshape(SQ, D).astype(bf16)
    Wqkv = jnp.concatenate([Wq, Wk, Wv], axis=1).astype(bf16)
    Wo = Wo.astype(bf16)

    inv = 10000.0 ** (-np.arange(0, DH, 2, dtype=np.float32) / DH)
    ang = np.arange(SQ, dtype=np.float32)[:, None] * inv[None, :]
    cos = np.tile(np.repeat(np.cos(ang), 2, axis=-1), (1, HQ))
    sin = np.tile(np.repeat(np.sin(ang), 2, axis=-1), (1, HQ))
    cosb = jnp.asarray(cos, dtype=bf16)
    sinb = jnp.asarray(sin, dtype=bf16)
    cosq = jnp.asarray(cos * SCALE, dtype=bf16)
    sinq = jnp.asarray(sin * SCALE, dtype=bf16)

    def body(x_ref, wqkv_ref, wo_ref,
             cosb_ref, sinb_ref, cosq_ref, sinq_ref,
             out_ref,
             qkv0, xall, psend, acc,
             x_send_sems, x_recv_sems, p_send_sems, p_recv_sems):
        me = lax.axis_index("i")

        barrier = pltpu.get_barrier_semaphore()
        for r in range(1, N_DEV):
            pl.semaphore_signal(
                barrier, inc=1,
                device_id=((me + r) % N_DEV,),
                device_id_type=pl.DeviceIdType.MESH,
            )
        pl.semaphore_wait(barrier, N_DEV - 1)

        def x_send(r):
            rdma = pltpu.make_async_remote_copy(
                src_ref=x_ref,
                dst_ref=xall.at[3 - r],
                send_sem=x_send_sems.at[r - 1],
                recv_sem=x_recv_sems.at[3 - r],
                device_id=((me + r) % N_DEV,),
                device_id_type=pl.DeviceIdType.MESH,
            )
            rdma.start()
            return rdma

        x_sends = [x_send(1), x_send(3)]

        def mm(a, b):
            return lax.dot_general(
                a, b, (((1,), (0,)), ((), ())),
                preferred_element_type=jnp.float32,
            )

        col_odd = lax.broadcasted_iota(jnp.int32, (SQ, D), 1) % 2 == 1

        def rope(t, c, sn):
            tb = t.astype(bf16)
            rr = pltpu.roll(tb, 1, 1)
            rl = pltpu.roll(tb, D - 1, 1)
            tr = jnp.where(col_odd, rr, -rl)
            return tb * c + tr * sn

        def qkv(xb):
            big = mm(xb, wqkv_ref[...])
            q = rope(big[:, :D], cosq_ref[...], sinq_ref[...])
            k = rope(big[:, D:2 * D], cosb_ref[...], sinb_ref[...])
            v = big[:, 2 * D:].astype(bf16)
            return q, k, v

        ones_col = jnp.ones((SQ, DH), bf16)

        def attn(q, k, v, h0, h1, r0=0, r1=SQ):
            ctxs = []
            for h in range(h0, h1):
                sl = slice(h * DH, (h + 1) * DH)
                s = lax.dot_general(
                    q[r0:r1, sl], k[:, sl], (((1,), (1,)), ((), ())),
                    preferred_element_type=jnp.float32,
                )
                e = jnp.exp(s).astype(bf16)
                ctxa = mm(e, jnp.concatenate([v[:, sl], ones_col], axis=1))
                ctxs.append((ctxa[:, :DH] / ctxa[:, DH:DH + 1]).astype(bf16))
            return mm(jnp.concatenate(ctxs, axis=1),
                      wo_ref[h0 * DH:h1 * DH, :])

        q0, k0, v0 = qkv(x_ref[...])
        qkv0[0, :, :] = q0
        qkv0[1, :, :] = k0
        qkv0[2, :, :] = v0
        x_sends.append(x_send(2))

        OWN_SPLIT = 2
        out_ref[0, :, :] = attn(qkv0[0], qkv0[1], qkv0[2], 0, OWN_SPLIT).astype(bf16)

        p_sends = []
        for s in (0, 1, 2):
            recv = pltpu.make_async_remote_copy(
                src_ref=x_ref,
                dst_ref=xall.at[s],
                send_sem=x_send_sems.at[0],
                recv_sem=x_recv_sems.at[s],
                device_id=(me,),
                device_id_type=pl.DeviceIdType.MESH,
            )
            recv.wait_recv()
            qs, ks, vs = qkv(xall[s])
            for c in range(2):
                r0, r1 = c * (SQ // 2), (c + 1) * (SQ // 2)
                psend[s, r0:r1, :] = attn(qs, ks, vs, 0, HQ, r0, r1).astype(bf16)
                rdma = pltpu.make_async_remote_copy(
                    src_ref=psend.at[s, r0:r1],
                    dst_ref=acc.at[2 - s, r0:r1],
                    send_sem=p_send_sems.at[s, c],
                    recv_sem=p_recv_sems.at[2 - s, c],
                    device_id=((me + s + 1) % N_DEV,),
                    device_id_type=pl.DeviceIdType.MESH,
                )
                rdma.start()
                p_sends.append(rdma)

        rest = attn(qkv0[0], qkv0[1], qkv0[2], OWN_SPLIT, HQ)

        for s in range(N_DEV - 1):
            for c in range(2):
                r0, r1 = c * (SQ // 2), (c + 1) * (SQ // 2)
                recv = pltpu.make_async_remote_copy(
                    src_ref=psend.at[0, r0:r1],
                    dst_ref=acc.at[s, r0:r1],
                    send_sem=p_send_sems.at[0, 0],
                    recv_sem=p_recv_sems.at[s, c],
                    device_id=(me,),
                    device_id_type=pl.DeviceIdType.MESH,
                )
                recv.wait_recv()

        out_ref[0, :, :] = (
            out_ref[0, :, :].astype(jnp.float32)
            + rest
            + acc[0].astype(jnp.float32)
            + acc[1].astype(jnp.float32)
            + acc[2].astype(jnp.float32)
        ).astype(bf16)

        for rdma in x_sends + p_sends:
            rdma.wait_send()

    return pl.pallas_call(
        body,
        out_shape=jax.ShapeDtypeStruct((1, SQ, D), jnp.bfloat16),
        in_specs=[pl.BlockSpec(memory_space=pltpu.VMEM)] * 7,
        out_specs=pl.BlockSpec(memory_space=pltpu.VMEM),
        scratch_shapes=[
            pltpu.VMEM((3, SQ, D), jnp.bfloat16),
            pltpu.VMEM((N_DEV - 1, SQ, D), jnp.bfloat16),
            pltpu.VMEM((N_DEV - 1, SQ, D), jnp.bfloat16),
            pltpu.VMEM((N_DEV - 1, SQ, D), jnp.bfloat16),
            pltpu.SemaphoreType.DMA((N_DEV - 1,)),
            pltpu.SemaphoreType.DMA((N_DEV - 1,)),
            pltpu.SemaphoreType.DMA((N_DEV - 1, 2)),
            pltpu.SemaphoreType.DMA((N_DEV - 1, 2)),
        ],
        compiler_params=pltpu.CompilerParams(
            collective_id=0, vmem_limit_bytes=100 * 1024 * 1024
        ),
    )(x, Wqkv, Wo, cosb, sinb, cosq, sinq)


# device time: 142804 ns/iter; 1.7471x vs baseline; 1.0178x over previous
import os

import jax
import jax.numpy as jnp
import numpy as np
from jax import lax
from jax.experimental import pallas as pl
from jax.experimental.pallas import tpu as pltpu

jax.config.update(
    "jax_compilation_cache_dir",
    os.path.join(os.path.dirname(os.path.abspath(__file__)), ".jaxcache"),
)
jax.config.update("jax_persistent_cache_min_compile_time_secs", 1.0)
jax.config.update("jax_persistent_cache_min_entry_size_bytes", 0)

N_DEV = 4
SQ = 1024
D = 1024
HQ = 8
DH = 128
SCALE = 0.08838834764831843


def kernel(x, Wq, Wk, Wv, Wo):
    bf16 = jnp.bfloat16
    Wqkv = jnp.concatenate([Wq, Wk, Wv], axis=1).astype(bf16)
    Wo = Wo.astype(bf16)

    inv = 10000.0 ** (-np.arange(0, DH, 2, dtype=np.float32) / DH)
    ang = np.arange(SQ, dtype=np.float32)[:, None] * inv[None, :]
    cos = np.tile(np.repeat(np.cos(ang), 2, axis=-1), (1, HQ))
    sin = np.tile(np.repeat(np.sin(ang), 2, axis=-1), (1, HQ))
    cosb = jnp.asarray(cos, dtype=bf16)
    sinb = jnp.asarray(sin, dtype=bf16)
    cosq = jnp.asarray(cos * SCALE, dtype=bf16)
    sinq = jnp.asarray(sin * SCALE, dtype=bf16)

    def body(x_ref, wqkv_ref, wo_ref,
             cosb_ref, sinb_ref, cosq_ref, sinq_ref,
             out_ref,
             xbf, qkv0, xall, psend, acc,
             x_send_sems, x_recv_sems, p_send_sems, p_recv_sems):
        me = lax.axis_index("i")

        xbf[...] = x_ref[0].astype(bf16)

        barrier = pltpu.get_barrier_semaphore()
        for r in range(1, N_DEV):
            pl.semaphore_signal(
                barrier, inc=1,
                device_id=((me + r) % N_DEV,),
                device_id_type=pl.DeviceIdType.MESH,
            )
        pl.semaphore_wait(barrier, N_DEV - 1)

        def x_send(r):
            rdma = pltpu.make_async_remote_copy(
                src_ref=xbf,
                dst_ref=xall.at[3 - r],
                send_sem=x_send_sems.at[r - 1],
                recv_sem=x_recv_sems.at[3 - r],
                device_id=((me + r) % N_DEV,),
                device_id_type=pl.DeviceIdType.MESH,
            )
            rdma.start()
            return rdma

        x_sends = [x_send(1), x_send(3)]

        def mm(a, b):
            return lax.dot_general(
                a, b, (((1,), (0,)), ((), ())),
                preferred_element_type=jnp.float32,
            )

        col_odd = lax.broadcasted_iota(jnp.int32, (SQ, D), 1) % 2 == 1

        def rope(t, c, sn):
            tb = t.astype(bf16)
            rr = pltpu.roll(tb, 1, 1)
            rl = pltpu.roll(tb, D - 1, 1)
            tr = jnp.where(col_odd, rr, -rl)
            return tb * c + tr * sn

        def qkv(xb):
            big = mm(xb, wqkv_ref[...])
            q = rope(big[:, :D], cosq_ref[...], sinq_ref[...])
            k = rope(big[:, D:2 * D], cosb_ref[...], sinb_ref[...])
            v = big[:, 2 * D:].astype(bf16)
            return q, k, v

        ones_col = jnp.ones((SQ, DH), bf16)

        def attn(q, k, v, h0, h1, r0=0, r1=SQ):
            ctxs = []
            for h in range(h0, h1):
                sl = slice(h * DH, (h + 1) * DH)
                s = lax.dot_general(
                    q[r0:r1, sl], k[:, sl], (((1,), (1,)), ((), ())),
                    preferred_element_type=jnp.float32,
                )
                e = jnp.exp(s).astype(bf16)
                ctxa = mm(e, jnp.concatenate([v[:, sl], ones_col], axis=1))
                ctxs.append((ctxa[:, :DH] / ctxa[:, DH:DH + 1]).astype(bf16))
            return mm(jnp.concatenate(ctxs, axis=1),
                      wo_ref[h0 * DH:h1 * DH, :])

        q0, k0, v0 = qkv(xbf[...])
        qkv0[0, :, :] = q0
        qkv0[1, :, :] = k0
        qkv0[2, :, :] = v0
        x_sends.append(x_send(2))

        OWN_SPLIT = 2
        out_ref[0, :, :] = attn(qkv0[0], qkv0[1], qkv0[2], 0, OWN_SPLIT).astype(bf16)

        p_sends = []
        for s in (0, 1, 2):
            recv = pltpu.make_async_remote_copy(
                src_ref=xbf,
                dst_ref=xall.at[s],
                send_sem=x_send_sems.at[0],
                recv_sem=x_recv_sems.at[s],
                device_id=(me,),
                device_id_type=pl.DeviceIdType.MESH,
            )
            recv.wait_recv()
            qs, ks, vs = qkv(xall[s])
            for c in range(2):
                r0, r1 = c * (SQ // 2), (c + 1) * (SQ // 2)
                psend[s, r0:r1, :] = attn(qs, ks, vs, 0, HQ, r0, r1).astype(bf16)
                rdma = pltpu.make_async_remote_copy(
                    src_ref=psend.at[s, r0:r1],
                    dst_ref=acc.at[2 - s, r0:r1],
                    send_sem=p_send_sems.at[s, c],
                    recv_sem=p_recv_sems.at[2 - s, c],
                    device_id=((me + s + 1) % N_DEV,),
                    device_id_type=pl.DeviceIdType.MESH,
                )
                rdma.start()
                p_sends.append(rdma)

        rest = attn(qkv0[0], qkv0[1], qkv0[2], OWN_SPLIT, HQ)

        for s in range(N_DEV - 1):
            for c in range(2):
                r0, r1 = c * (SQ // 2), (c + 1) * (SQ // 2)
                recv = pltpu.make_async_remote_copy(
                    src_ref=psend.at[0, r0:r1],
                    dst_ref=acc.at[s, r0:r1],
                    send_sem=p_send_sems.at[0, 0],
                    recv_sem=p_recv_sems.at[s, c],
                    device_id=(me,),
                    device_id_type=pl.DeviceIdType.MESH,
                )
                recv.wait_recv()

        out_ref[0, :, :] = (
            out_ref[0, :, :].astype(jnp.float32)
            + rest
            + acc[0].astype(jnp.float32)
            + acc[1].astype(jnp.float32)
            + acc[2].astype(jnp.float32)
        ).astype(bf16)

        for rdma in x_sends + p_sends:
            rdma.wait_send()

    return pl.pallas_call(
        body,
        out_shape=jax.ShapeDtypeStruct((1, SQ, D), jnp.bfloat16),
        in_specs=[pl.BlockSpec(memory_space=pltpu.VMEM)] * 7,
        out_specs=pl.BlockSpec(memory_space=pltpu.VMEM),
        scratch_shapes=[
            pltpu.VMEM((SQ, D), jnp.bfloat16),
            pltpu.VMEM((3, SQ, D), jnp.bfloat16),
            pltpu.VMEM((N_DEV - 1, SQ, D), jnp.bfloat16),
            pltpu.VMEM((N_DEV - 1, SQ, D), jnp.bfloat16),
            pltpu.VMEM((N_DEV - 1, SQ, D), jnp.bfloat16),
            pltpu.SemaphoreType.DMA((N_DEV - 1,)),
            pltpu.SemaphoreType.DMA((N_DEV - 1,)),
            pltpu.SemaphoreType.DMA((N_DEV - 1, 2)),
            pltpu.SemaphoreType.DMA((N_DEV - 1, 2)),
        ],
        compiler_params=pltpu.CompilerParams(
            collective_id=0, vmem_limit_bytes=100 * 1024 * 1024
        ),
    )(x, Wqkv, Wo, cosb, sinb, cosq, sinq)
